# Initial kernel scaffold; baseline (speedup 1.0000x reference)
import jax
import jax.numpy as jnp
from jax.experimental import pallas as pl


def kernel(x_user, x_item, edge_index_user_item, edge_index_item_user, l0_ui_Wl, l0_ui_bl, l0_ui_Wr, l0_iu_Wl, l0_iu_bl, l0_iu_Wr, l1_ui_Wl, l1_ui_bl, l1_ui_Wr, l1_iu_Wl, l1_iu_bl, l1_iu_Wr, Wn, bn, Wf, bf):
    raise NotImplementedError("write your pallas kernel here")


# speedup vs baseline: 1.9037x; 1.9037x over previous
import functools

import jax
import jax.numpy as jnp
from jax import lax
from jax.experimental import pallas as pl
from jax.experimental.pallas import tpu as pltpu
from jax.experimental.pallas import tpu_sc as plsc

N = 50000
D = 128
E = 600000
DC = 16
NSLICE = D // DC
NSUB = 16
EPT = 38912
EPAD = NSUB * EPT
SUB = 128
CHUNK = 1024
NSTREAM = CHUNK // SUB
NCHUNK = EPT // CHUNK
ACC_ROWS = 50048
ZERO_PT = ACC_ROWS // NSUB
OUT_PT = ZERO_PT
IDX_ROWS_PT = EPT // SUB


def _sc_body(with_cnt, *refs):
    it = iter(refs)

    def take(n):
        return [next(it) for _ in range(n)]

    xu = take(NSLICE)
    src_ui, dst_ui = take(2)
    xi = take(NSLICE)
    src_iu, dst_iu = take(2)
    zeros_h, ones_h = take(2)
    sui = take(NSLICE)
    siu = take(NSLICE)
    cui, ciu = take(2) if with_cnt else (None, None)
    sidx_v, didx_v, rows_v, acc, sem = take(5)

    c = lax.axis_index("c")
    s = lax.axis_index("s")

    def zero_acc():
        pltpu.sync_copy(zeros_h.at[pl.ds(s * ZERO_PT, ZERO_PT)],
                        acc.at[pl.ds(s * ZERO_PT, ZERO_PT)])

    def scatter_chunk():
        for j in range(NSTREAM):
            pltpu.sync_copy(rows_v.at[pl.ds(j * SUB, SUB)],
                            acc.at[didx_v.at[j]], add=True)

    def run_rel(xs, src_h, dst_h, outs, cout):
        for dc in range(NSLICE):
            zero_acc()
            plsc.subcore_barrier()

            def chunk(i, carry):
                row0 = s * IDX_ROWS_PT + i * NSTREAM
                pltpu.sync_copy(src_h.at[pl.ds(row0, NSTREAM)], sidx_v)
                pltpu.sync_copy(dst_h.at[pl.ds(row0, NSTREAM)], didx_v)
                descs = [pltpu.async_copy(xs[dc].at[sidx_v.at[j]],
                                          rows_v.at[pl.ds(j * SUB, SUB)], sem)
                         for j in range(NSTREAM)]
                for d_ in descs:
                    d_.wait()
                scatter_chunk()
                return carry

            lax.fori_loop(0, NCHUNK, chunk, 0)
            plsc.subcore_barrier()
            pltpu.sync_copy(acc.at[pl.ds(s * OUT_PT, OUT_PT)],
                            outs[dc].at[pl.ds(s * OUT_PT, OUT_PT)])
            plsc.subcore_barrier()

        if cout is not None:
            zero_acc()
            pltpu.sync_copy(ones_h, rows_v)
            plsc.subcore_barrier()

            def cchunk(i, carry):
                row0 = s * IDX_ROWS_PT + i * NSTREAM
                pltpu.sync_copy(dst_h.at[pl.ds(row0, NSTREAM)], didx_v)
                scatter_chunk()
                return carry

            lax.fori_loop(0, NCHUNK, cchunk, 0)
            plsc.subcore_barrier()
            pltpu.sync_copy(acc.at[pl.ds(s * OUT_PT, OUT_PT)],
                            cout.at[pl.ds(s * OUT_PT, OUT_PT)])

    @pl.when(c == 0)
    def _():
        run_rel(xu, src_ui, dst_ui, sui, cui)

    @pl.when(c == 1)
    def _():
        run_rel(xi, src_iu, dst_iu, siu, ciu)


def _make_sc(with_cnt):
    mesh = plsc.VectorSubcoreMesh(core_axis_name="c", subcore_axis_name="s")
    n_out = 2 * NSLICE + (2 if with_cnt else 0)
    out_type = tuple(jax.ShapeDtypeStruct((ACC_ROWS, DC), jnp.float32)
                     for _ in range(n_out))
    scratch = [
        pltpu.VMEM((NSTREAM, SUB), jnp.int32),
        pltpu.VMEM((NSTREAM, SUB), jnp.int32),
        pltpu.VMEM((CHUNK, DC), jnp.float32),
        pltpu.VMEM_SHARED((ACC_ROWS, DC), jnp.float32),
        pltpu.SemaphoreType.DMA,
    ]
    return pl.kernel(functools.partial(_sc_body, with_cnt),
                     out_type=out_type, mesh=mesh, scratch_types=scratch,
                     compiler_params=pltpu.CompilerParams(
                         use_tc_tiling_on_sc=False,
                         internal_scratch_in_bytes=128 * 1024))


_sc_l0 = _make_sc(True)
_sc_l1 = _make_sc(False)


BLK = 512
GRID = (N + BLK - 1) // BLK


def _combine_body(s_ref, cnt_ref, x_ref, wl_ref, bl_ref, wr_ref, o_ref):
    cnt = jnp.maximum(cnt_ref[...][:, 0:1], 1.0)
    aggr = s_ref[...] / cnt
    out = (jnp.dot(aggr, wl_ref[...], preferred_element_type=jnp.float32)
           + bl_ref[...][None, :]
           + jnp.dot(x_ref[...], wr_ref[...], preferred_element_type=jnp.float32))
    o_ref[...] = jnp.maximum(out, 0.0)


def _combine(s, cnt32, x, wl, bl, wr):
    return pl.pallas_call(
        _combine_body,
        grid=(GRID,),
        in_specs=[pl.BlockSpec((BLK, D), lambda i: (i, 0)),
                  pl.BlockSpec((BLK, DC), lambda i: (i, 0)),
                  pl.BlockSpec((BLK, D), lambda i: (i, 0)),
                  pl.BlockSpec((D, D), lambda i: (0, 0)),
                  pl.BlockSpec((D,), lambda i: (0,)),
                  pl.BlockSpec((D, D), lambda i: (0, 0))],
        out_specs=pl.BlockSpec((BLK, D), lambda i: (i, 0)),
        out_shape=jax.ShapeDtypeStruct((N, D), jnp.float32),
    )(s, cnt32, x, wl, bl, wr)


def _readout_body(xu_ref, xi_ref, wn_ref, bn_ref, su_ref, si_ref):
    i = pl.program_id(0)
    rows = i * BLK + lax.broadcasted_iota(jnp.int32, (BLK, 1), 0)
    mask = rows < N
    bn = bn_ref[...][None, :]
    hu = jnp.maximum(jnp.dot(xu_ref[...], wn_ref[...],
                             preferred_element_type=jnp.float32) + bn, 0.0)
    hi = jnp.maximum(jnp.dot(xi_ref[...], wn_ref[...],
                             preferred_element_type=jnp.float32) + bn, 0.0)
    su = jnp.sum(jnp.where(mask, hu, 0.0), axis=0, keepdims=True)
    si = jnp.sum(jnp.where(mask, hi, 0.0), axis=0, keepdims=True)

    @pl.when(i == 0)
    def _():
        su_ref[...] = jnp.zeros_like(su_ref)
        si_ref[...] = jnp.zeros_like(si_ref)

    su_ref[...] += su
    si_ref[...] += si


def _readout_sums(xu, xi, wn, bn):
    return pl.pallas_call(
        _readout_body,
        grid=(GRID,),
        in_specs=[pl.BlockSpec((BLK, D), lambda i: (i, 0)),
                  pl.BlockSpec((BLK, D), lambda i: (i, 0)),
                  pl.BlockSpec((D, D), lambda i: (0, 0)),
                  pl.BlockSpec((D,), lambda i: (0,))],
        out_specs=[pl.BlockSpec((1, D), lambda i: (0, 0)),
                   pl.BlockSpec((1, D), lambda i: (0, 0))],
        out_shape=[jax.ShapeDtypeStruct((1, D), jnp.float32),
                   jax.ShapeDtypeStruct((1, D), jnp.float32)],
    )(xu, xi, wn, bn)


def _final_body(su_ref, si_ref, wf_ref, bf_ref, o_ref):
    fu = su_ref[...] / float(N)
    fi = si_ref[...] / float(N)
    z = (jnp.dot(fu, wf_ref[0:D, :], preferred_element_type=jnp.float32)
         + jnp.dot(fi, wf_ref[D:2 * D, :], preferred_element_type=jnp.float32)
         + bf_ref[...][None, :])
    z = z - jnp.max(z, axis=1, keepdims=True)
    e = jnp.exp(z)
    o_ref[...] = e / jnp.sum(e, axis=1, keepdims=True)


def _final(su, si, wf, bf):
    return pl.pallas_call(
        _final_body,
        out_shape=jax.ShapeDtypeStruct((1, wf.shape[1]), jnp.float32),
    )(su, si, wf, bf)


def _pad_edges(ei):
    src = jnp.concatenate([ei[0], jnp.zeros((EPAD - E,), jnp.int32)])
    dst = jnp.concatenate([ei[1], jnp.full((EPAD - E,), N, jnp.int32)])
    return src.reshape(EPAD // SUB, SUB), dst.reshape(EPAD // SUB, SUB)


def _xsl(x):
    return [x[:, k * DC:(k + 1) * DC] for k in range(NSLICE)]


def kernel(x_user, x_item, edge_index_user_item, edge_index_item_user,
           l0_ui_Wl, l0_ui_bl, l0_ui_Wr, l0_iu_Wl, l0_iu_bl, l0_iu_Wr,
           l1_ui_Wl, l1_ui_bl, l1_ui_Wr, l1_iu_Wl, l1_iu_bl, l1_iu_Wr,
           Wn, bn, Wf, bf):
    src_ui, dst_ui = _pad_edges(edge_index_user_item)
    src_iu, dst_iu = _pad_edges(edge_index_item_user)
    zeros_h = jnp.zeros((ACC_ROWS, DC), jnp.float32)
    ones_h = jnp.ones((CHUNK, DC), jnp.float32)

    outs0 = _sc_l0(*_xsl(x_user), src_ui, dst_ui,
                   *_xsl(x_item), src_iu, dst_iu, zeros_h, ones_h)
    sui = jnp.concatenate([o[:N] for o in outs0[0:NSLICE]], axis=1)
    siu = jnp.concatenate([o[:N] for o in outs0[NSLICE:2 * NSLICE]], axis=1)
    cnt_item32 = outs0[2 * NSLICE][:N]
    cnt_user32 = outs0[2 * NSLICE + 1][:N]

    xi1 = _combine(sui, cnt_item32, x_item, l0_ui_Wl, l0_ui_bl, l0_ui_Wr)
    xu1 = _combine(siu, cnt_user32, x_user, l0_iu_Wl, l0_iu_bl, l0_iu_Wr)

    outs1 = _sc_l1(*_xsl(xu1), src_ui, dst_ui,
                   *_xsl(xi1), src_iu, dst_iu, zeros_h, ones_h)
    sui1 = jnp.concatenate([o[:N] for o in outs1[0:NSLICE]], axis=1)
    siu1 = jnp.concatenate([o[:N] for o in outs1[NSLICE:2 * NSLICE]], axis=1)

    xi2 = _combine(sui1, cnt_item32, xi1, l1_ui_Wl, l1_ui_bl, l1_ui_Wr)
    xu2 = _combine(siu1, cnt_user32, xu1, l1_iu_Wl, l1_iu_bl, l1_iu_Wr)

    su, si = _readout_sums(xu2, xi2, Wn, bn)
    out = _final(su, si, Wf, bf)
    return jnp.reshape(out, (bf.shape[0],))


# speedup vs baseline: 1.9373x; 1.0176x over previous
import functools

import jax
import jax.numpy as jnp
from jax import lax
from jax.experimental import pallas as pl
from jax.experimental.pallas import tpu as pltpu
from jax.experimental.pallas import tpu_sc as plsc

N = 50000
D = 128
E = 600000
DC = 16
NSLICE = D // DC
NSUB = 16
EPT = 38912
EPAD = NSUB * EPT
SUB = 128
CHUNK = 1024
NSTREAM = CHUNK // SUB
NCHUNK = EPT // CHUNK
ACC_ROWS = 50048
ZERO_PT = ACC_ROWS // NSUB
OUT_PT = ZERO_PT
IDX_ROWS_PT = EPT // SUB


def _sc_body(with_cnt, *refs):
    it = iter(refs)

    def take(n):
        return [next(it) for _ in range(n)]

    xu = take(NSLICE)
    src_ui, dst_ui = take(2)
    xi = take(NSLICE)
    src_iu, dst_iu = take(2)
    zeros_h, ones_h = take(2)
    sui = take(NSLICE)
    siu = take(NSLICE)
    cui, ciu = take(2) if with_cnt else (None, None)
    sidx_v, didx_v, rows_v, acc, sem = take(5)

    c = lax.axis_index("c")
    s = lax.axis_index("s")

    def zero_acc():
        pltpu.sync_copy(zeros_h.at[pl.ds(s * ZERO_PT, ZERO_PT)],
                        acc.at[pl.ds(s * ZERO_PT, ZERO_PT)])

    def run_rel(xs, src_h, dst_h, outs, cout):
        for dc in range(NSLICE):
            zero_acc()
            plsc.subcore_barrier()

            def chunk(i, carry):
                off = s * EPT + i * CHUNK
                pltpu.sync_copy(src_h.at[pl.ds(off, CHUNK)], sidx_v)
                pltpu.sync_copy(dst_h.at[pl.ds(off, CHUNK)], didx_v)
                pltpu.async_copy(xs[dc].at[sidx_v], rows_v, sem).wait()
                pltpu.sync_copy(rows_v, acc.at[didx_v], add=True)
                return carry

            lax.fori_loop(0, NCHUNK, chunk, 0)
            plsc.subcore_barrier()
            pltpu.sync_copy(acc.at[pl.ds(s * OUT_PT, OUT_PT)],
                            outs[dc].at[pl.ds(s * OUT_PT, OUT_PT)])
            plsc.subcore_barrier()

        if cout is not None:
            zero_acc()
            pltpu.sync_copy(ones_h, rows_v)
            plsc.subcore_barrier()

            def cchunk(i, carry):
                off = s * EPT + i * CHUNK
                pltpu.sync_copy(dst_h.at[pl.ds(off, CHUNK)], didx_v)
                pltpu.sync_copy(rows_v, acc.at[didx_v], add=True)
                return carry

            lax.fori_loop(0, NCHUNK, cchunk, 0)
            plsc.subcore_barrier()
            pltpu.sync_copy(acc.at[pl.ds(s * OUT_PT, OUT_PT)],
                            cout.at[pl.ds(s * OUT_PT, OUT_PT)])

    @pl.when(c == 0)
    def _():
        run_rel(xu, src_ui, dst_ui, sui, cui)

    @pl.when(c == 1)
    def _():
        run_rel(xi, src_iu, dst_iu, siu, ciu)


def _make_sc(with_cnt):
    mesh = plsc.VectorSubcoreMesh(core_axis_name="c", subcore_axis_name="s")
    n_out = 2 * NSLICE + (2 if with_cnt else 0)
    out_type = tuple(jax.ShapeDtypeStruct((ACC_ROWS, DC), jnp.float32)
                     for _ in range(n_out))
    scratch = [
        pltpu.VMEM((CHUNK,), jnp.int32),
        pltpu.VMEM((CHUNK,), jnp.int32),
        pltpu.VMEM((CHUNK, DC), jnp.float32),
        pltpu.VMEM_SHARED((ACC_ROWS, DC), jnp.float32),
        pltpu.SemaphoreType.DMA,
    ]
    return pl.kernel(functools.partial(_sc_body, with_cnt),
                     out_type=out_type, mesh=mesh, scratch_types=scratch,
                     compiler_params=pltpu.CompilerParams(
                         use_tc_tiling_on_sc=False,
                         internal_scratch_in_bytes=128 * 1024))


_sc_l0 = _make_sc(True)
_sc_l1 = _make_sc(False)


BLK = 512
GRID = (N + BLK - 1) // BLK


def _combine_body(s_ref, cnt_ref, x_ref, wl_ref, bl_ref, wr_ref, o_ref):
    cnt = jnp.maximum(cnt_ref[...][:, 0:1], 1.0)
    aggr = s_ref[...] / cnt
    out = (jnp.dot(aggr, wl_ref[...], preferred_element_type=jnp.float32)
           + bl_ref[...][None, :]
           + jnp.dot(x_ref[...], wr_ref[...], preferred_element_type=jnp.float32))
    o_ref[...] = jnp.maximum(out, 0.0)


def _combine(s, cnt32, x, wl, bl, wr):
    return pl.pallas_call(
        _combine_body,
        grid=(GRID,),
        in_specs=[pl.BlockSpec((BLK, D), lambda i: (i, 0)),
                  pl.BlockSpec((BLK, DC), lambda i: (i, 0)),
                  pl.BlockSpec((BLK, D), lambda i: (i, 0)),
                  pl.BlockSpec((D, D), lambda i: (0, 0)),
                  pl.BlockSpec((D,), lambda i: (0,)),
                  pl.BlockSpec((D, D), lambda i: (0, 0))],
        out_specs=pl.BlockSpec((BLK, D), lambda i: (i, 0)),
        out_shape=jax.ShapeDtypeStruct((N, D), jnp.float32),
    )(s, cnt32, x, wl, bl, wr)


def _readout_body(xu_ref, xi_ref, wn_ref, bn_ref, su_ref, si_ref):
    i = pl.program_id(0)
    rows = i * BLK + lax.broadcasted_iota(jnp.int32, (BLK, 1), 0)
    mask = rows < N
    bn = bn_ref[...][None, :]
    hu = jnp.maximum(jnp.dot(xu_ref[...], wn_ref[...],
                             preferred_element_type=jnp.float32) + bn, 0.0)
    hi = jnp.maximum(jnp.dot(xi_ref[...], wn_ref[...],
                             preferred_element_type=jnp.float32) + bn, 0.0)
    su = jnp.sum(jnp.where(mask, hu, 0.0), axis=0, keepdims=True)
    si = jnp.sum(jnp.where(mask, hi, 0.0), axis=0, keepdims=True)

    @pl.when(i == 0)
    def _():
        su_ref[...] = jnp.zeros_like(su_ref)
        si_ref[...] = jnp.zeros_like(si_ref)

    su_ref[...] += su
    si_ref[...] += si


def _readout_sums(xu, xi, wn, bn):
    return pl.pallas_call(
        _readout_body,
        grid=(GRID,),
        in_specs=[pl.BlockSpec((BLK, D), lambda i: (i, 0)),
                  pl.BlockSpec((BLK, D), lambda i: (i, 0)),
                  pl.BlockSpec((D, D), lambda i: (0, 0)),
                  pl.BlockSpec((D,), lambda i: (0,))],
        out_specs=[pl.BlockSpec((1, D), lambda i: (0, 0)),
                   pl.BlockSpec((1, D), lambda i: (0, 0))],
        out_shape=[jax.ShapeDtypeStruct((1, D), jnp.float32),
                   jax.ShapeDtypeStruct((1, D), jnp.float32)],
    )(xu, xi, wn, bn)


def _final_body(su_ref, si_ref, wf_ref, bf_ref, o_ref):
    fu = su_ref[...] / float(N)
    fi = si_ref[...] / float(N)
    z = (jnp.dot(fu, wf_ref[0:D, :], preferred_element_type=jnp.float32)
         + jnp.dot(fi, wf_ref[D:2 * D, :], preferred_element_type=jnp.float32)
         + bf_ref[...][None, :])
    z = z - jnp.max(z, axis=1, keepdims=True)
    e = jnp.exp(z)
    o_ref[...] = e / jnp.sum(e, axis=1, keepdims=True)


def _final(su, si, wf, bf):
    return pl.pallas_call(
        _final_body,
        out_shape=jax.ShapeDtypeStruct((1, wf.shape[1]), jnp.float32),
    )(su, si, wf, bf)


def _pad_edges(ei):
    src = jnp.concatenate([ei[0], jnp.zeros((EPAD - E,), jnp.int32)])
    dst = jnp.concatenate([ei[1], jnp.full((EPAD - E,), N, jnp.int32)])
    return src, dst


def _xsl(x):
    return [x[:, k * DC:(k + 1) * DC] for k in range(NSLICE)]


def kernel(x_user, x_item, edge_index_user_item, edge_index_item_user,
           l0_ui_Wl, l0_ui_bl, l0_ui_Wr, l0_iu_Wl, l0_iu_bl, l0_iu_Wr,
           l1_ui_Wl, l1_ui_bl, l1_ui_Wr, l1_iu_Wl, l1_iu_bl, l1_iu_Wr,
           Wn, bn, Wf, bf):
    src_ui, dst_ui = _pad_edges(edge_index_user_item)
    src_iu, dst_iu = _pad_edges(edge_index_item_user)
    zeros_h = jnp.zeros((ACC_ROWS, DC), jnp.float32)
    ones_h = jnp.ones((CHUNK, DC), jnp.float32)

    outs0 = _sc_l0(*_xsl(x_user), src_ui, dst_ui,
                   *_xsl(x_item), src_iu, dst_iu, zeros_h, ones_h)
    sui = jnp.concatenate([o[:N] for o in outs0[0:NSLICE]], axis=1)
    siu = jnp.concatenate([o[:N] for o in outs0[NSLICE:2 * NSLICE]], axis=1)
    cnt_item32 = outs0[2 * NSLICE][:N]
    cnt_user32 = outs0[2 * NSLICE + 1][:N]

    xi1 = _combine(sui, cnt_item32, x_item, l0_ui_Wl, l0_ui_bl, l0_ui_Wr)
    xu1 = _combine(siu, cnt_user32, x_user, l0_iu_Wl, l0_iu_bl, l0_iu_Wr)

    outs1 = _sc_l1(*_xsl(xu1), src_ui, dst_ui,
                   *_xsl(xi1), src_iu, dst_iu, zeros_h, ones_h)
    sui1 = jnp.concatenate([o[:N] for o in outs1[0:NSLICE]], axis=1)
    siu1 = jnp.concatenate([o[:N] for o in outs1[NSLICE:2 * NSLICE]], axis=1)

    xi2 = _combine(sui1, cnt_item32, xi1, l1_ui_Wl, l1_ui_bl, l1_ui_Wr)
    xu2 = _combine(siu1, cnt_user32, xu1, l1_iu_Wl, l1_iu_bl, l1_iu_Wr)

    su, si = _readout_sums(xu2, xi2, Wn, bn)
    out = _final(su, si, Wf, bf)
    return jnp.reshape(out, (bf.shape[0],))


# speedup vs baseline: 2.9138x; 1.5041x over previous
import functools

import jax
import jax.numpy as jnp
from jax import lax
from jax.experimental import pallas as pl
from jax.experimental.pallas import tpu as pltpu
from jax.experimental.pallas import tpu_sc as plsc

N = 50000
D = 128
E = 600000
DC = 16
NSLICE = D // DC
NSUB = 16
EPT = 38912
EPAD = NSUB * EPT
SUB = 128
CHUNK = 1024
NSTREAM = CHUNK // SUB
NCHUNK = EPT // CHUNK
ACC_ROWS = 50048
ZERO_PT = ACC_ROWS // NSUB
OUT_PT = ZERO_PT
IDX_ROWS_PT = EPT // SUB


def _sc_body(with_cnt, *refs):
    it = iter(refs)

    def take(n):
        return [next(it) for _ in range(n)]

    (xu, src_ui, dst_ui, xi, src_iu, dst_iu, zeros_h, ones_h,
     sui, siu) = take(10)
    cui, ciu = take(2) if with_cnt else (None, None)
    sidx_v, didx_v, rows_v, acc, sem = take(5)

    c = lax.axis_index("c")
    s = lax.axis_index("s")

    def zero_acc():
        pltpu.sync_copy(zeros_h.at[pl.ds(s * ZERO_PT, ZERO_PT)],
                        acc.at[pl.ds(s * ZERO_PT, ZERO_PT)])

    def run_rel(xs, src_h, dst_h, outs, cout):
        for dc in range(NSLICE):
            zero_acc()
            plsc.subcore_barrier()

            def chunk(i, carry):
                off = s * EPT + i * CHUNK
                pltpu.sync_copy(src_h.at[pl.ds(dc * EPAD + off, CHUNK)],
                                sidx_v)
                pltpu.sync_copy(dst_h.at[pl.ds(off, CHUNK)], didx_v)
                pltpu.async_copy(xs.at[sidx_v], rows_v, sem).wait()
                pltpu.sync_copy(rows_v, acc.at[didx_v], add=True)
                return carry

            lax.fori_loop(0, NCHUNK, chunk, 0)
            plsc.subcore_barrier()
            pltpu.sync_copy(acc.at[pl.ds(s * OUT_PT, OUT_PT)],
                            outs.at[pl.ds(s * OUT_PT, OUT_PT),
                                    pl.ds(dc * DC, DC)])
            plsc.subcore_barrier()

        if cout is not None:
            zero_acc()
            pltpu.sync_copy(ones_h, rows_v)
            plsc.subcore_barrier()

            def cchunk(i, carry):
                off = s * EPT + i * CHUNK
                pltpu.sync_copy(dst_h.at[pl.ds(off, CHUNK)], didx_v)
                pltpu.sync_copy(rows_v, acc.at[didx_v], add=True)
                return carry

            lax.fori_loop(0, NCHUNK, cchunk, 0)
            plsc.subcore_barrier()
            pltpu.sync_copy(acc.at[pl.ds(s * OUT_PT, OUT_PT)],
                            cout.at[pl.ds(s * OUT_PT, OUT_PT), pl.ds(0, DC)])

    @pl.when(c == 0)
    def _():
        run_rel(xu, src_ui, dst_ui, sui, cui)

    @pl.when(c == 1)
    def _():
        run_rel(xi, src_iu, dst_iu, siu, ciu)


def _make_sc(with_cnt):
    mesh = plsc.VectorSubcoreMesh(core_axis_name="c", subcore_axis_name="s")
    n_out = 2 + (2 if with_cnt else 0)
    out_type = tuple(jax.ShapeDtypeStruct((ACC_ROWS, D), jnp.float32)
                     for _ in range(n_out))
    scratch = [
        pltpu.VMEM((CHUNK,), jnp.int32),
        pltpu.VMEM((CHUNK,), jnp.int32),
        pltpu.VMEM((CHUNK, DC), jnp.float32),
        pltpu.VMEM_SHARED((ACC_ROWS, DC), jnp.float32),
        pltpu.SemaphoreType.DMA,
    ]
    return pl.kernel(functools.partial(_sc_body, with_cnt),
                     out_type=out_type, mesh=mesh, scratch_types=scratch,
                     compiler_params=pltpu.CompilerParams(
                         use_tc_tiling_on_sc=False,
                         internal_scratch_in_bytes=128 * 1024))


_sc_l0 = _make_sc(True)
_sc_l1 = _make_sc(False)


BLK = 512
GRID = (N + BLK - 1) // BLK


def _combine_body(s_ref, cnt_ref, x_ref, wl_ref, bl_ref, wr_ref, o_ref):
    cnt = jnp.maximum(cnt_ref[...][:, 0:1], 1.0)
    aggr = s_ref[...] / cnt
    out = (jnp.dot(aggr, wl_ref[...], preferred_element_type=jnp.float32)
           + bl_ref[...][None, :]
           + jnp.dot(x_ref[...], wr_ref[...], preferred_element_type=jnp.float32))
    o_ref[...] = jnp.maximum(out, 0.0)


def _combine(s, cnt32, x, wl, bl, wr):
    return pl.pallas_call(
        _combine_body,
        grid=(GRID,),
        in_specs=[pl.BlockSpec((BLK, D), lambda i: (i, 0)),
                  pl.BlockSpec((BLK, D), lambda i: (i, 0)),
                  pl.BlockSpec((BLK, D), lambda i: (i, 0)),
                  pl.BlockSpec((D, D), lambda i: (0, 0)),
                  pl.BlockSpec((D,), lambda i: (0,)),
                  pl.BlockSpec((D, D), lambda i: (0, 0))],
        out_specs=pl.BlockSpec((BLK, D), lambda i: (i, 0)),
        out_shape=jax.ShapeDtypeStruct((N, D), jnp.float32),
    )(s, cnt32, x, wl, bl, wr)


def _readout_body(xu_ref, xi_ref, wn_ref, bn_ref, su_ref, si_ref):
    i = pl.program_id(0)
    rows = i * BLK + lax.broadcasted_iota(jnp.int32, (BLK, 1), 0)
    mask = rows < N
    bn = bn_ref[...][None, :]
    hu = jnp.maximum(jnp.dot(xu_ref[...], wn_ref[...],
                             preferred_element_type=jnp.float32) + bn, 0.0)
    hi = jnp.maximum(jnp.dot(xi_ref[...], wn_ref[...],
                             preferred_element_type=jnp.float32) + bn, 0.0)
    su = jnp.sum(jnp.where(mask, hu, 0.0), axis=0, keepdims=True)
    si = jnp.sum(jnp.where(mask, hi, 0.0), axis=0, keepdims=True)

    @pl.when(i == 0)
    def _():
        su_ref[...] = jnp.zeros_like(su_ref)
        si_ref[...] = jnp.zeros_like(si_ref)

    su_ref[...] += su
    si_ref[...] += si


def _readout_sums(xu, xi, wn, bn):
    return pl.pallas_call(
        _readout_body,
        grid=(GRID,),
        in_specs=[pl.BlockSpec((BLK, D), lambda i: (i, 0)),
                  pl.BlockSpec((BLK, D), lambda i: (i, 0)),
                  pl.BlockSpec((D, D), lambda i: (0, 0)),
                  pl.BlockSpec((D,), lambda i: (0,))],
        out_specs=[pl.BlockSpec((1, D), lambda i: (0, 0)),
                   pl.BlockSpec((1, D), lambda i: (0, 0))],
        out_shape=[jax.ShapeDtypeStruct((1, D), jnp.float32),
                   jax.ShapeDtypeStruct((1, D), jnp.float32)],
    )(xu, xi, wn, bn)


def _final_body(su_ref, si_ref, wf_ref, bf_ref, o_ref):
    fu = su_ref[...] / float(N)
    fi = si_ref[...] / float(N)
    z = (jnp.dot(fu, wf_ref[0:D, :], preferred_element_type=jnp.float32)
         + jnp.dot(fi, wf_ref[D:2 * D, :], preferred_element_type=jnp.float32)
         + bf_ref[...][None, :])
    z = z - jnp.max(z, axis=1, keepdims=True)
    e = jnp.exp(z)
    o_ref[...] = e / jnp.sum(e, axis=1, keepdims=True)


def _final(su, si, wf, bf):
    return pl.pallas_call(
        _final_body,
        out_shape=jax.ShapeDtypeStruct((1, wf.shape[1]), jnp.float32),
    )(su, si, wf, bf)


def _pad_edges(ei):
    src = jnp.concatenate([ei[0], jnp.zeros((EPAD - E,), jnp.int32)])
    srcdc = (src[None, :] * NSLICE
             + jnp.arange(NSLICE, dtype=jnp.int32)[:, None]).reshape(-1)
    dst = jnp.concatenate([ei[1], jnp.full((EPAD - E,), N, jnp.int32)])
    return srcdc, dst


def _flat(x):
    return x.reshape(N * NSLICE, DC)


def kernel(x_user, x_item, edge_index_user_item, edge_index_item_user,
           l0_ui_Wl, l0_ui_bl, l0_ui_Wr, l0_iu_Wl, l0_iu_bl, l0_iu_Wr,
           l1_ui_Wl, l1_ui_bl, l1_ui_Wr, l1_iu_Wl, l1_iu_bl, l1_iu_Wr,
           Wn, bn, Wf, bf):
    src_ui, dst_ui = _pad_edges(edge_index_user_item)
    src_iu, dst_iu = _pad_edges(edge_index_item_user)
    zeros_h = jnp.zeros((ACC_ROWS, DC), jnp.float32)
    ones_h = jnp.ones((CHUNK, DC), jnp.float32)

    sui, siu, cnt_item32, cnt_user32 = _sc_l0(
        _flat(x_user), src_ui, dst_ui, _flat(x_item), src_iu, dst_iu,
        zeros_h, ones_h)

    xi1 = _combine(sui, cnt_item32, x_item, l0_ui_Wl, l0_ui_bl, l0_ui_Wr)
    xu1 = _combine(siu, cnt_user32, x_user, l0_iu_Wl, l0_iu_bl, l0_iu_Wr)

    sui1, siu1 = _sc_l1(
        _flat(xu1), src_ui, dst_ui, _flat(xi1), src_iu, dst_iu,
        zeros_h, ones_h)

    xi2 = _combine(sui1, cnt_item32, xi1, l1_ui_Wl, l1_ui_bl, l1_ui_Wr)
    xu2 = _combine(siu1, cnt_user32, xu1, l1_iu_Wl, l1_iu_bl, l1_iu_Wr)

    su, si = _readout_sums(xu2, xi2, Wn, bn)
    out = _final(su, si, Wf, bf)
    return jnp.reshape(out, (bf.shape[0],))


# speedup vs baseline: 3.2340x; 1.1099x over previous
import functools

import jax
import jax.numpy as jnp
from jax import lax
from jax.experimental import pallas as pl
from jax.experimental.pallas import tpu as pltpu
from jax.experimental.pallas import tpu_sc as plsc

N = 50000
D = 128
E = 600000
DC = 16
NSLICE = D // DC
NSUB = 16
EPT = 38912
EPAD = NSUB * EPT
SUB = 128
CHUNK = 1024
NSTREAM = CHUNK // SUB
NCHUNK = EPT // CHUNK
ACC_ROWS = 50048
ZERO_PT = ACC_ROWS // NSUB
OUT_PT = ZERO_PT
IDX_ROWS_PT = EPT // SUB


def _sc_body(with_cnt, *refs):
    it = iter(refs)

    def take(n):
        return [next(it) for _ in range(n)]

    (xu, src_ui, dst_ui, xi, src_iu, dst_iu, zeros_h, ones_h,
     sui, siu) = take(10)
    cui, ciu = take(2) if with_cnt else (None, None)
    sidx_v, didx_v, rows0, rows1, acc, sem0, sem1 = take(7)

    c = lax.axis_index("c")
    s = lax.axis_index("s")

    def zero_acc():
        pltpu.sync_copy(zeros_h.at[pl.ds(s * ZERO_PT, ZERO_PT)],
                        acc.at[pl.ds(s * ZERO_PT, ZERO_PT)])

    def gather(xs, half, buf, sem):
        return pltpu.async_copy(
            xs.at[sidx_v.at[pl.ds(half * CHUNK, CHUNK)]], buf, sem)

    def gather_wait(xs, buf, sem):
        pltpu.make_async_copy(xs.at[pl.ds(0, CHUNK)], buf, sem).wait()

    def scatter(half, buf):
        pltpu.sync_copy(buf, acc.at[didx_v.at[pl.ds(half * CHUNK, CHUNK)]],
                        add=True)

    def run_rel(xs, src_h, dst_h, outs, cout):
        for dc in range(NSLICE):
            zero_acc()
            plsc.subcore_barrier()

            def chunk2(i, carry):
                off = s * EPT + 2 * i * CHUNK
                pltpu.sync_copy(src_h.at[pl.ds(dc * EPAD + off, 2 * CHUNK)],
                                sidx_v)
                pltpu.sync_copy(dst_h.at[pl.ds(off, 2 * CHUNK)], didx_v)
                gather(xs, 0, rows0, sem0)
                gather(xs, 1, rows1, sem1)
                gather_wait(xs, rows0, sem0)
                scatter(0, rows0)
                gather_wait(xs, rows1, sem1)
                scatter(1, rows1)
                return carry

            lax.fori_loop(0, NCHUNK // 2, chunk2, 0)
            plsc.subcore_barrier()
            pltpu.sync_copy(acc.at[pl.ds(s * OUT_PT, OUT_PT)],
                            outs.at[pl.ds(s * OUT_PT, OUT_PT),
                                    pl.ds(dc * DC, DC)])
            plsc.subcore_barrier()

        if cout is not None:
            zero_acc()
            pltpu.sync_copy(ones_h, rows0)
            plsc.subcore_barrier()

            def cchunk(i, carry):
                off = s * EPT + 2 * i * CHUNK
                pltpu.sync_copy(dst_h.at[pl.ds(off, 2 * CHUNK)], didx_v)
                scatter(0, rows0)
                scatter(1, rows0)
                return carry

            lax.fori_loop(0, NCHUNK // 2, cchunk, 0)
            plsc.subcore_barrier()
            pltpu.sync_copy(acc.at[pl.ds(s * OUT_PT, OUT_PT)],
                            cout.at[pl.ds(s * OUT_PT, OUT_PT), pl.ds(0, DC)])

    @pl.when(c == 0)
    def _():
        run_rel(xu, src_ui, dst_ui, sui, cui)

    @pl.when(c == 1)
    def _():
        run_rel(xi, src_iu, dst_iu, siu, ciu)


def _make_sc(with_cnt):
    mesh = plsc.VectorSubcoreMesh(core_axis_name="c", subcore_axis_name="s")
    n_out = 2 + (2 if with_cnt else 0)
    out_type = tuple(jax.ShapeDtypeStruct((ACC_ROWS, D), jnp.float32)
                     for _ in range(n_out))
    scratch = [
        pltpu.VMEM((2 * CHUNK,), jnp.int32),
        pltpu.VMEM((2 * CHUNK,), jnp.int32),
        pltpu.VMEM((CHUNK, DC), jnp.float32),
        pltpu.VMEM((CHUNK, DC), jnp.float32),
        pltpu.VMEM_SHARED((ACC_ROWS, DC), jnp.float32),
        pltpu.SemaphoreType.DMA,
        pltpu.SemaphoreType.DMA,
    ]
    return pl.kernel(functools.partial(_sc_body, with_cnt),
                     out_type=out_type, mesh=mesh, scratch_types=scratch,
                     compiler_params=pltpu.CompilerParams(
                         use_tc_tiling_on_sc=False,
                         internal_scratch_in_bytes=128 * 1024))


_sc_l0 = _make_sc(True)
_sc_l1 = _make_sc(False)


BLK = 512
GRID = (N + BLK - 1) // BLK


def _combine_body(s_ref, cnt_ref, x_ref, wl_ref, bl_ref, wr_ref, o_ref):
    cnt = jnp.maximum(cnt_ref[...][:, 0:1], 1.0)
    aggr = s_ref[...] / cnt
    out = (jnp.dot(aggr, wl_ref[...], preferred_element_type=jnp.float32)
           + bl_ref[...][None, :]
           + jnp.dot(x_ref[...], wr_ref[...], preferred_element_type=jnp.float32))
    o_ref[...] = jnp.maximum(out, 0.0)


def _combine(s, cnt32, x, wl, bl, wr):
    return pl.pallas_call(
        _combine_body,
        grid=(GRID,),
        in_specs=[pl.BlockSpec((BLK, D), lambda i: (i, 0)),
                  pl.BlockSpec((BLK, D), lambda i: (i, 0)),
                  pl.BlockSpec((BLK, D), lambda i: (i, 0)),
                  pl.BlockSpec((D, D), lambda i: (0, 0)),
                  pl.BlockSpec((D,), lambda i: (0,)),
                  pl.BlockSpec((D, D), lambda i: (0, 0))],
        out_specs=pl.BlockSpec((BLK, D), lambda i: (i, 0)),
        out_shape=jax.ShapeDtypeStruct((N, D), jnp.float32),
    )(s, cnt32, x, wl, bl, wr)


def _readout_body(xu_ref, xi_ref, wn_ref, bn_ref, su_ref, si_ref):
    i = pl.program_id(0)
    rows = i * BLK + lax.broadcasted_iota(jnp.int32, (BLK, 1), 0)
    mask = rows < N
    bn = bn_ref[...][None, :]
    hu = jnp.maximum(jnp.dot(xu_ref[...], wn_ref[...],
                             preferred_element_type=jnp.float32) + bn, 0.0)
    hi = jnp.maximum(jnp.dot(xi_ref[...], wn_ref[...],
                             preferred_element_type=jnp.float32) + bn, 0.0)
    su = jnp.sum(jnp.where(mask, hu, 0.0), axis=0, keepdims=True)
    si = jnp.sum(jnp.where(mask, hi, 0.0), axis=0, keepdims=True)

    @pl.when(i == 0)
    def _():
        su_ref[...] = jnp.zeros_like(su_ref)
        si_ref[...] = jnp.zeros_like(si_ref)

    su_ref[...] += su
    si_ref[...] += si


def _readout_sums(xu, xi, wn, bn):
    return pl.pallas_call(
        _readout_body,
        grid=(GRID,),
        in_specs=[pl.BlockSpec((BLK, D), lambda i: (i, 0)),
                  pl.BlockSpec((BLK, D), lambda i: (i, 0)),
                  pl.BlockSpec((D, D), lambda i: (0, 0)),
                  pl.BlockSpec((D,), lambda i: (0,))],
        out_specs=[pl.BlockSpec((1, D), lambda i: (0, 0)),
                   pl.BlockSpec((1, D), lambda i: (0, 0))],
        out_shape=[jax.ShapeDtypeStruct((1, D), jnp.float32),
                   jax.ShapeDtypeStruct((1, D), jnp.float32)],
    )(xu, xi, wn, bn)


def _final_body(su_ref, si_ref, wf_ref, bf_ref, o_ref):
    fu = su_ref[...] / float(N)
    fi = si_ref[...] / float(N)
    z = (jnp.dot(fu, wf_ref[0:D, :], preferred_element_type=jnp.float32)
         + jnp.dot(fi, wf_ref[D:2 * D, :], preferred_element_type=jnp.float32)
         + bf_ref[...][None, :])
    z = z - jnp.max(z, axis=1, keepdims=True)
    e = jnp.exp(z)
    o_ref[...] = e / jnp.sum(e, axis=1, keepdims=True)


def _final(su, si, wf, bf):
    return pl.pallas_call(
        _final_body,
        out_shape=jax.ShapeDtypeStruct((1, wf.shape[1]), jnp.float32),
    )(su, si, wf, bf)


def _pad_edges(ei):
    src = jnp.concatenate([ei[0], jnp.zeros((EPAD - E,), jnp.int32)])
    srcdc = (src[None, :] * NSLICE
             + jnp.arange(NSLICE, dtype=jnp.int32)[:, None]).reshape(-1)
    dst = jnp.concatenate([ei[1], jnp.full((EPAD - E,), N, jnp.int32)])
    return srcdc, dst


def _flat(x):
    return x.reshape(N * NSLICE, DC)


def kernel(x_user, x_item, edge_index_user_item, edge_index_item_user,
           l0_ui_Wl, l0_ui_bl, l0_ui_Wr, l0_iu_Wl, l0_iu_bl, l0_iu_Wr,
           l1_ui_Wl, l1_ui_bl, l1_ui_Wr, l1_iu_Wl, l1_iu_bl, l1_iu_Wr,
           Wn, bn, Wf, bf):
    src_ui, dst_ui = _pad_edges(edge_index_user_item)
    src_iu, dst_iu = _pad_edges(edge_index_item_user)
    zeros_h = jnp.zeros((ACC_ROWS, DC), jnp.float32)
    ones_h = jnp.ones((CHUNK, DC), jnp.float32)

    sui, siu, cnt_item32, cnt_user32 = _sc_l0(
        _flat(x_user), src_ui, dst_ui, _flat(x_item), src_iu, dst_iu,
        zeros_h, ones_h)

    xi1 = _combine(sui, cnt_item32, x_item, l0_ui_Wl, l0_ui_bl, l0_ui_Wr)
    xu1 = _combine(siu, cnt_user32, x_user, l0_iu_Wl, l0_iu_bl, l0_iu_Wr)

    sui1, siu1 = _sc_l1(
        _flat(xu1), src_ui, dst_ui, _flat(xi1), src_iu, dst_iu,
        zeros_h, ones_h)

    xi2 = _combine(sui1, cnt_item32, xi1, l1_ui_Wl, l1_ui_bl, l1_ui_Wr)
    xu2 = _combine(siu1, cnt_user32, xu1, l1_iu_Wl, l1_iu_bl, l1_iu_Wr)

    su, si = _readout_sums(xu2, xi2, Wn, bn)
    out = _final(su, si, Wf, bf)
    return jnp.reshape(out, (bf.shape[0],))


# speedup vs baseline: 3.4491x; 1.0665x over previous
import functools

import jax
import jax.numpy as jnp
from jax import lax
from jax.experimental import pallas as pl
from jax.experimental.pallas import tpu as pltpu
from jax.experimental.pallas import tpu_sc as plsc

N = 50000
D = 128
E = 600000
DC = 32
NSLICE = D // DC
NSUB = 16
EPT = 38400
EPAD = NSUB * EPT
CHUNK = 192
NCHUNK = EPT // CHUNK
ACC_ROWS = 50048
ZERO_PT = ACC_ROWS // NSUB
OUT_PT = ZERO_PT


def _sc_body(with_cnt, *refs):
    it = iter(refs)

    def take(n):
        return [next(it) for _ in range(n)]

    (xu, src_ui, dst_ui, xi, src_iu, dst_iu, zeros_h, ones_h,
     sui, siu) = take(10)
    cui, ciu = take(2) if with_cnt else (None, None)
    sidx_v, didx_v, rows0, rows1, acc, sem0, sem1 = take(7)

    c = lax.axis_index("c")
    s = lax.axis_index("s")

    def zero_acc():
        pltpu.sync_copy(zeros_h.at[pl.ds(s * ZERO_PT, ZERO_PT)],
                        acc.at[pl.ds(s * ZERO_PT, ZERO_PT)])

    def gather(xs, half, buf, sem):
        return pltpu.async_copy(
            xs.at[sidx_v.at[pl.ds(half * CHUNK, CHUNK)]], buf, sem)

    def gather_wait(xs, buf, sem):
        pltpu.make_async_copy(xs.at[pl.ds(0, CHUNK)], buf, sem).wait()

    def scatter(half, buf):
        pltpu.sync_copy(buf, acc.at[didx_v.at[pl.ds(half * CHUNK, CHUNK)]],
                        add=True)

    def run_rel(xs, src_h, dst_h, outs, cout):
        for dc in range(NSLICE):
            zero_acc()
            plsc.subcore_barrier()

            def chunk2(i, carry):
                off = s * EPT + 2 * i * CHUNK
                pltpu.sync_copy(src_h.at[pl.ds(dc * EPAD + off, 2 * CHUNK)],
                                sidx_v)
                pltpu.sync_copy(dst_h.at[pl.ds(off, 2 * CHUNK)], didx_v)
                gather(xs, 0, rows0, sem0)
                gather(xs, 1, rows1, sem1)
                gather_wait(xs, rows0, sem0)
                scatter(0, rows0)
                gather_wait(xs, rows1, sem1)
                scatter(1, rows1)
                return carry

            lax.fori_loop(0, NCHUNK // 2, chunk2, 0)
            plsc.subcore_barrier()
            pltpu.sync_copy(acc.at[pl.ds(s * OUT_PT, OUT_PT)],
                            outs.at[pl.ds(s * OUT_PT, OUT_PT),
                                    pl.ds(dc * DC, DC)])
            plsc.subcore_barrier()

        if cout is not None:
            zero_acc()
            pltpu.sync_copy(ones_h, rows0)
            plsc.subcore_barrier()

            def cchunk(i, carry):
                off = s * EPT + 2 * i * CHUNK
                pltpu.sync_copy(dst_h.at[pl.ds(off, 2 * CHUNK)], didx_v)
                scatter(0, rows0)
                scatter(1, rows0)
                return carry

            lax.fori_loop(0, NCHUNK // 2, cchunk, 0)
            plsc.subcore_barrier()
            pltpu.sync_copy(acc.at[pl.ds(s * OUT_PT, OUT_PT)],
                            cout.at[pl.ds(s * OUT_PT, OUT_PT), pl.ds(0, DC)])

    @pl.when(c == 0)
    def _():
        run_rel(xu, src_ui, dst_ui, sui, cui)

    @pl.when(c == 1)
    def _():
        run_rel(xi, src_iu, dst_iu, siu, ciu)


def _make_sc(with_cnt):
    mesh = plsc.VectorSubcoreMesh(core_axis_name="c", subcore_axis_name="s")
    n_out = 2 + (2 if with_cnt else 0)
    out_type = tuple(jax.ShapeDtypeStruct((ACC_ROWS, D), jnp.float32)
                     for _ in range(n_out))
    scratch = [
        pltpu.VMEM((2 * CHUNK,), jnp.int32),
        pltpu.VMEM((2 * CHUNK,), jnp.int32),
        pltpu.VMEM((CHUNK, DC), jnp.float32),
        pltpu.VMEM((CHUNK, DC), jnp.float32),
        pltpu.VMEM_SHARED((ACC_ROWS, DC), jnp.float32),
        pltpu.SemaphoreType.DMA,
        pltpu.SemaphoreType.DMA,
    ]
    return pl.kernel(functools.partial(_sc_body, with_cnt),
                     out_type=out_type, mesh=mesh, scratch_types=scratch,
                     compiler_params=pltpu.CompilerParams(
                         use_tc_tiling_on_sc=False,
                         internal_scratch_in_bytes=128 * 1024))


_sc_l0 = _make_sc(True)
_sc_l1 = _make_sc(False)


BLK = 512
GRID = (N + BLK - 1) // BLK


def _combine_body(s_ref, cnt_ref, x_ref, wl_ref, bl_ref, wr_ref, o_ref):
    cnt = jnp.maximum(cnt_ref[...][:, 0:1], 1.0)
    aggr = s_ref[...] / cnt
    out = (jnp.dot(aggr, wl_ref[...], preferred_element_type=jnp.float32)
           + bl_ref[...][None, :]
           + jnp.dot(x_ref[...], wr_ref[...], preferred_element_type=jnp.float32))
    o_ref[...] = jnp.maximum(out, 0.0)


def _combine(s, cnt32, x, wl, bl, wr):
    return pl.pallas_call(
        _combine_body,
        grid=(GRID,),
        in_specs=[pl.BlockSpec((BLK, D), lambda i: (i, 0)),
                  pl.BlockSpec((BLK, D), lambda i: (i, 0)),
                  pl.BlockSpec((BLK, D), lambda i: (i, 0)),
                  pl.BlockSpec((D, D), lambda i: (0, 0)),
                  pl.BlockSpec((D,), lambda i: (0,)),
                  pl.BlockSpec((D, D), lambda i: (0, 0))],
        out_specs=pl.BlockSpec((BLK, D), lambda i: (i, 0)),
        out_shape=jax.ShapeDtypeStruct((N, D), jnp.float32),
    )(s, cnt32, x, wl, bl, wr)


def _readout_body(xu_ref, xi_ref, wn_ref, bn_ref, su_ref, si_ref):
    i = pl.program_id(0)
    rows = i * BLK + lax.broadcasted_iota(jnp.int32, (BLK, 1), 0)
    mask = rows < N
    bn = bn_ref[...][None, :]
    hu = jnp.maximum(jnp.dot(xu_ref[...], wn_ref[...],
                             preferred_element_type=jnp.float32) + bn, 0.0)
    hi = jnp.maximum(jnp.dot(xi_ref[...], wn_ref[...],
                             preferred_element_type=jnp.float32) + bn, 0.0)
    su = jnp.sum(jnp.where(mask, hu, 0.0), axis=0, keepdims=True)
    si = jnp.sum(jnp.where(mask, hi, 0.0), axis=0, keepdims=True)

    @pl.when(i == 0)
    def _():
        su_ref[...] = jnp.zeros_like(su_ref)
        si_ref[...] = jnp.zeros_like(si_ref)

    su_ref[...] += su
    si_ref[...] += si


def _readout_sums(xu, xi, wn, bn):
    return pl.pallas_call(
        _readout_body,
        grid=(GRID,),
        in_specs=[pl.BlockSpec((BLK, D), lambda i: (i, 0)),
                  pl.BlockSpec((BLK, D), lambda i: (i, 0)),
                  pl.BlockSpec((D, D), lambda i: (0, 0)),
                  pl.BlockSpec((D,), lambda i: (0,))],
        out_specs=[pl.BlockSpec((1, D), lambda i: (0, 0)),
                   pl.BlockSpec((1, D), lambda i: (0, 0))],
        out_shape=[jax.ShapeDtypeStruct((1, D), jnp.float32),
                   jax.ShapeDtypeStruct((1, D), jnp.float32)],
    )(xu, xi, wn, bn)


def _final_body(su_ref, si_ref, wf_ref, bf_ref, o_ref):
    fu = su_ref[...] / float(N)
    fi = si_ref[...] / float(N)
    z = (jnp.dot(fu, wf_ref[0:D, :], preferred_element_type=jnp.float32)
         + jnp.dot(fi, wf_ref[D:2 * D, :], preferred_element_type=jnp.float32)
         + bf_ref[...][None, :])
    z = z - jnp.max(z, axis=1, keepdims=True)
    e = jnp.exp(z)
    o_ref[...] = e / jnp.sum(e, axis=1, keepdims=True)


def _final(su, si, wf, bf):
    return pl.pallas_call(
        _final_body,
        out_shape=jax.ShapeDtypeStruct((1, wf.shape[1]), jnp.float32),
    )(su, si, wf, bf)


def _pad_edges(ei):
    src = jnp.concatenate([ei[0], jnp.zeros((EPAD - E,), jnp.int32)])
    srcdc = (src[None, :] * NSLICE
             + jnp.arange(NSLICE, dtype=jnp.int32)[:, None]).reshape(-1)
    dst = jnp.concatenate([ei[1], jnp.full((EPAD - E,), N, jnp.int32)])
    return srcdc, dst


def _flat(x):
    return x.reshape(N * NSLICE, DC)


def kernel(x_user, x_item, edge_index_user_item, edge_index_item_user,
           l0_ui_Wl, l0_ui_bl, l0_ui_Wr, l0_iu_Wl, l0_iu_bl, l0_iu_Wr,
           l1_ui_Wl, l1_ui_bl, l1_ui_Wr, l1_iu_Wl, l1_iu_bl, l1_iu_Wr,
           Wn, bn, Wf, bf):
    src_ui, dst_ui = _pad_edges(edge_index_user_item)
    src_iu, dst_iu = _pad_edges(edge_index_item_user)
    zeros_h = jnp.zeros((ACC_ROWS, DC), jnp.float32)
    ones_h = jnp.ones((CHUNK, DC), jnp.float32)

    sui, siu, cnt_item32, cnt_user32 = _sc_l0(
        _flat(x_user), src_ui, dst_ui, _flat(x_item), src_iu, dst_iu,
        zeros_h, ones_h)

    xi1 = _combine(sui, cnt_item32, x_item, l0_ui_Wl, l0_ui_bl, l0_ui_Wr)
    xu1 = _combine(siu, cnt_user32, x_user, l0_iu_Wl, l0_iu_bl, l0_iu_Wr)

    sui1, siu1 = _sc_l1(
        _flat(xu1), src_ui, dst_ui, _flat(xi1), src_iu, dst_iu,
        zeros_h, ones_h)

    xi2 = _combine(sui1, cnt_item32, xi1, l1_ui_Wl, l1_ui_bl, l1_ui_Wr)
    xu2 = _combine(siu1, cnt_user32, xu1, l1_iu_Wl, l1_iu_bl, l1_iu_Wr)

    su, si = _readout_sums(xu2, xi2, Wn, bn)
    out = _final(su, si, Wf, bf)
    return jnp.reshape(out, (bf.shape[0],))


# speedup vs baseline: 3.4903x; 1.0120x over previous
import functools

import jax
import jax.numpy as jnp
from jax import lax
from jax.experimental import pallas as pl
from jax.experimental.pallas import tpu as pltpu
from jax.experimental.pallas import tpu_sc as plsc

N = 50000
D = 128
E = 600000
DC = 32
NSLICE = D // DC
NSUB = 16
EPT = 38400
EPAD = NSUB * EPT
CHUNK = 192
NCHUNK = EPT // CHUNK
ACC_ROWS = 50048
ZERO_PT = ACC_ROWS // NSUB
OUT_PT = ZERO_PT


def _sc_body(with_cnt, *refs):
    it = iter(refs)

    def take(n):
        return [next(it) for _ in range(n)]

    (xu, src_ui, dst_ui, xi, src_iu, dst_iu, zeros_h, ones_h,
     sui, siu) = take(10)
    cui, ciu = take(2) if with_cnt else (None, None)
    sidx_v, didx_v, rows0, rows1, acc, sem0, sem1, sems0, sems1 = take(9)

    c = lax.axis_index("c")
    s = lax.axis_index("s")

    def zero_acc():
        pltpu.sync_copy(zeros_h.at[pl.ds(s * ZERO_PT, ZERO_PT)],
                        acc.at[pl.ds(s * ZERO_PT, ZERO_PT)])

    def gather(xs, half, buf, sem):
        return pltpu.async_copy(
            xs.at[sidx_v.at[pl.ds(half * CHUNK, CHUNK)]], buf, sem)

    def gather_wait(xs, buf, sem):
        pltpu.make_async_copy(xs.at[pl.ds(0, CHUNK)], buf, sem).wait()

    def scatter(half, buf, sem):
        return pltpu.async_copy(
            buf, acc.at[didx_v.at[pl.ds(half * CHUNK, CHUNK)]], sem,
            add=True)

    def scatter_wait(buf, sem):
        pltpu.make_async_copy(buf, acc.at[pl.ds(0, CHUNK)], sem).wait()

    def run_rel(xs, src_h, dst_h, outs, cout):
        for dc in range(NSLICE):
            zero_acc()
            plsc.subcore_barrier()

            def chunk2(i, carry):
                off = s * EPT + 2 * i * CHUNK
                pltpu.sync_copy(src_h.at[pl.ds(dc * EPAD + off, 2 * CHUNK)],
                                sidx_v)
                pltpu.sync_copy(dst_h.at[pl.ds(off, 2 * CHUNK)], didx_v)
                gather(xs, 0, rows0, sem0)
                gather(xs, 1, rows1, sem1)
                gather_wait(xs, rows0, sem0)
                scatter(0, rows0, sems0)
                gather_wait(xs, rows1, sem1)
                scatter(1, rows1, sems1)
                scatter_wait(rows0, sems0)
                scatter_wait(rows1, sems1)
                return carry

            lax.fori_loop(0, NCHUNK // 2, chunk2, 0)
            plsc.subcore_barrier()
            pltpu.sync_copy(acc.at[pl.ds(s * OUT_PT, OUT_PT)],
                            outs.at[pl.ds(s * OUT_PT, OUT_PT),
                                    pl.ds(dc * DC, DC)])
            plsc.subcore_barrier()

        if cout is not None:
            zero_acc()
            pltpu.sync_copy(ones_h, rows0)
            plsc.subcore_barrier()

            def cchunk(i, carry):
                off = s * EPT + 2 * i * CHUNK
                pltpu.sync_copy(dst_h.at[pl.ds(off, 2 * CHUNK)], didx_v)
                scatter(0, rows0, sems0)
                scatter(1, rows0, sems1)
                scatter_wait(rows0, sems0)
                scatter_wait(rows0, sems1)
                return carry

            lax.fori_loop(0, NCHUNK // 2, cchunk, 0)
            plsc.subcore_barrier()
            pltpu.sync_copy(acc.at[pl.ds(s * OUT_PT, OUT_PT)],
                            cout.at[pl.ds(s * OUT_PT, OUT_PT), pl.ds(0, DC)])

    @pl.when(c == 0)
    def _():
        run_rel(xu, src_ui, dst_ui, sui, cui)

    @pl.when(c == 1)
    def _():
        run_rel(xi, src_iu, dst_iu, siu, ciu)


def _make_sc(with_cnt):
    mesh = plsc.VectorSubcoreMesh(core_axis_name="c", subcore_axis_name="s")
    n_out = 2 + (2 if with_cnt else 0)
    out_type = tuple(jax.ShapeDtypeStruct((ACC_ROWS, D), jnp.float32)
                     for _ in range(n_out))
    scratch = [
        pltpu.VMEM((2 * CHUNK,), jnp.int32),
        pltpu.VMEM((2 * CHUNK,), jnp.int32),
        pltpu.VMEM((CHUNK, DC), jnp.float32),
        pltpu.VMEM((CHUNK, DC), jnp.float32),
        pltpu.VMEM_SHARED((ACC_ROWS, DC), jnp.float32),
        pltpu.SemaphoreType.DMA,
        pltpu.SemaphoreType.DMA,
        pltpu.SemaphoreType.DMA,
        pltpu.SemaphoreType.DMA,
    ]
    return pl.kernel(functools.partial(_sc_body, with_cnt),
                     out_type=out_type, mesh=mesh, scratch_types=scratch,
                     compiler_params=pltpu.CompilerParams(
                         use_tc_tiling_on_sc=False,
                         internal_scratch_in_bytes=128 * 1024))


_sc_l0 = _make_sc(True)
_sc_l1 = _make_sc(False)


BLK = 512
GRID = (N + BLK - 1) // BLK


def _combine_body(s_ref, cnt_ref, x_ref, wl_ref, bl_ref, wr_ref, o_ref):
    cnt = jnp.maximum(cnt_ref[...][:, 0:1], 1.0)
    aggr = s_ref[...] / cnt
    out = (jnp.dot(aggr, wl_ref[...], preferred_element_type=jnp.float32)
           + bl_ref[...][None, :]
           + jnp.dot(x_ref[...], wr_ref[...], preferred_element_type=jnp.float32))
    o_ref[...] = jnp.maximum(out, 0.0)


def _combine(s, cnt32, x, wl, bl, wr):
    return pl.pallas_call(
        _combine_body,
        grid=(GRID,),
        in_specs=[pl.BlockSpec((BLK, D), lambda i: (i, 0)),
                  pl.BlockSpec((BLK, D), lambda i: (i, 0)),
                  pl.BlockSpec((BLK, D), lambda i: (i, 0)),
                  pl.BlockSpec((D, D), lambda i: (0, 0)),
                  pl.BlockSpec((D,), lambda i: (0,)),
                  pl.BlockSpec((D, D), lambda i: (0, 0))],
        out_specs=pl.BlockSpec((BLK, D), lambda i: (i, 0)),
        out_shape=jax.ShapeDtypeStruct((N, D), jnp.float32),
    )(s, cnt32, x, wl, bl, wr)


def _readout_body(xu_ref, xi_ref, wn_ref, bn_ref, su_ref, si_ref):
    i = pl.program_id(0)
    rows = i * BLK + lax.broadcasted_iota(jnp.int32, (BLK, 1), 0)
    mask = rows < N
    bn = bn_ref[...][None, :]
    hu = jnp.maximum(jnp.dot(xu_ref[...], wn_ref[...],
                             preferred_element_type=jnp.float32) + bn, 0.0)
    hi = jnp.maximum(jnp.dot(xi_ref[...], wn_ref[...],
                             preferred_element_type=jnp.float32) + bn, 0.0)
    su = jnp.sum(jnp.where(mask, hu, 0.0), axis=0, keepdims=True)
    si = jnp.sum(jnp.where(mask, hi, 0.0), axis=0, keepdims=True)

    @pl.when(i == 0)
    def _():
        su_ref[...] = jnp.zeros_like(su_ref)
        si_ref[...] = jnp.zeros_like(si_ref)

    su_ref[...] += su
    si_ref[...] += si


def _readout_sums(xu, xi, wn, bn):
    return pl.pallas_call(
        _readout_body,
        grid=(GRID,),
        in_specs=[pl.BlockSpec((BLK, D), lambda i: (i, 0)),
                  pl.BlockSpec((BLK, D), lambda i: (i, 0)),
                  pl.BlockSpec((D, D), lambda i: (0, 0)),
                  pl.BlockSpec((D,), lambda i: (0,))],
        out_specs=[pl.BlockSpec((1, D), lambda i: (0, 0)),
                   pl.BlockSpec((1, D), lambda i: (0, 0))],
        out_shape=[jax.ShapeDtypeStruct((1, D), jnp.float32),
                   jax.ShapeDtypeStruct((1, D), jnp.float32)],
    )(xu, xi, wn, bn)


def _final_body(su_ref, si_ref, wf_ref, bf_ref, o_ref):
    fu = su_ref[...] / float(N)
    fi = si_ref[...] / float(N)
    z = (jnp.dot(fu, wf_ref[0:D, :], preferred_element_type=jnp.float32)
         + jnp.dot(fi, wf_ref[D:2 * D, :], preferred_element_type=jnp.float32)
         + bf_ref[...][None, :])
    z = z - jnp.max(z, axis=1, keepdims=True)
    e = jnp.exp(z)
    o_ref[...] = e / jnp.sum(e, axis=1, keepdims=True)


def _final(su, si, wf, bf):
    return pl.pallas_call(
        _final_body,
        out_shape=jax.ShapeDtypeStruct((1, wf.shape[1]), jnp.float32),
    )(su, si, wf, bf)


def _pad_edges(ei):
    src = jnp.concatenate([ei[0], jnp.zeros((EPAD - E,), jnp.int32)])
    srcdc = (src[None, :] * NSLICE
             + jnp.arange(NSLICE, dtype=jnp.int32)[:, None]).reshape(-1)
    dst = jnp.concatenate([ei[1], jnp.full((EPAD - E,), N, jnp.int32)])
    return srcdc, dst


def _flat(x):
    return x.reshape(N * NSLICE, DC)


def kernel(x_user, x_item, edge_index_user_item, edge_index_item_user,
           l0_ui_Wl, l0_ui_bl, l0_ui_Wr, l0_iu_Wl, l0_iu_bl, l0_iu_Wr,
           l1_ui_Wl, l1_ui_bl, l1_ui_Wr, l1_iu_Wl, l1_iu_bl, l1_iu_Wr,
           Wn, bn, Wf, bf):
    src_ui, dst_ui = _pad_edges(edge_index_user_item)
    src_iu, dst_iu = _pad_edges(edge_index_item_user)
    zeros_h = jnp.zeros((ACC_ROWS, DC), jnp.float32)
    ones_h = jnp.ones((CHUNK, DC), jnp.float32)

    sui, siu, cnt_item32, cnt_user32 = _sc_l0(
        _flat(x_user), src_ui, dst_ui, _flat(x_item), src_iu, dst_iu,
        zeros_h, ones_h)

    xi1 = _combine(sui, cnt_item32, x_item, l0_ui_Wl, l0_ui_bl, l0_ui_Wr)
    xu1 = _combine(siu, cnt_user32, x_user, l0_iu_Wl, l0_iu_bl, l0_iu_Wr)

    sui1, siu1 = _sc_l1(
        _flat(xu1), src_ui, dst_ui, _flat(xi1), src_iu, dst_iu,
        zeros_h, ones_h)

    xi2 = _combine(sui1, cnt_item32, xi1, l1_ui_Wl, l1_ui_bl, l1_ui_Wr)
    xu2 = _combine(siu1, cnt_user32, xu1, l1_iu_Wl, l1_iu_bl, l1_iu_Wr)

    su, si = _readout_sums(xu2, xi2, Wn, bn)
    out = _final(su, si, Wf, bf)
    return jnp.reshape(out, (bf.shape[0],))


# speedup vs baseline: 3.9346x; 1.1273x over previous
import functools

import jax
import jax.numpy as jnp
from jax import lax
from jax.experimental import pallas as pl
from jax.experimental.pallas import tpu as pltpu
from jax.experimental.pallas import tpu_sc as plsc

N = 50000
D = 128
E = 600000
DC = 32
NSLICE = D // DC
NSUB = 16
EPT = 38400
EPAD = NSUB * EPT
CHUNK = 192
NCHUNK = EPT // CHUNK
ACC_ROWS = 50048
ZERO_PT = ACC_ROWS // NSUB
OUT_PT = ZERO_PT


def _sc_body(with_cnt, *refs):
    it = iter(refs)

    def take(n):
        return [next(it) for _ in range(n)]

    (xu, pk_ui, xi, pk_iu, zeros_h, ones_h, sui, siu) = take(8)
    cui, ciu = take(2) if with_cnt else (None, None)
    (idxa, idxb, rows0, rows1, acc,
     sem0, sem1, sems0, sems1, sema, semb) = take(11)

    c = lax.axis_index("c")
    s = lax.axis_index("s")

    def zero_acc():
        pltpu.sync_copy(zeros_h.at[pl.ds(s * ZERO_PT, ZERO_PT)],
                        acc.at[pl.ds(s * ZERO_PT, ZERO_PT)])

    def gather(xs, ibuf, half, buf, sem):
        return pltpu.async_copy(
            xs.at[ibuf.at[pl.ds(half * CHUNK, CHUNK)]], buf, sem)

    def gather_wait(xs, buf, sem):
        pltpu.make_async_copy(xs.at[pl.ds(0, CHUNK)], buf, sem).wait()

    def scatter(ibuf, half, buf, sem):
        return pltpu.async_copy(
            buf, acc.at[ibuf.at[pl.ds((2 + half) * CHUNK, CHUNK)]], sem,
            add=True)

    def scatter_wait(buf, sem):
        pltpu.make_async_copy(buf, acc.at[pl.ds(0, CHUNK)], sem).wait()

    NPAIR = NCHUNK // 2

    def idx_load(pk_h, dc, p, buf, sem):
        base = ((dc * NSUB + s) * NPAIR + p) * 4 * CHUNK
        return pltpu.async_copy(pk_h.at[pl.ds(base, 4 * CHUNK)], buf, sem)

    def idx_wait(pk_h, buf, sem):
        pltpu.make_async_copy(pk_h.at[pl.ds(0, 4 * CHUNK)], buf, sem).wait()

    def process_pair(xs, ibuf):
        gather(xs, ibuf, 0, rows0, sem0)
        gather(xs, ibuf, 1, rows1, sem1)
        gather_wait(xs, rows0, sem0)
        scatter(ibuf, 0, rows0, sems0)
        gather_wait(xs, rows1, sem1)
        scatter(ibuf, 1, rows1, sems1)
        scatter_wait(rows0, sems0)
        scatter_wait(rows1, sems1)

    def run_rel(xs, pk_h, outs, cout):
        for dc in range(NSLICE):
            zero_acc()
            plsc.subcore_barrier()
            idx_load(pk_h, dc, 0, idxa, sema)

            def pair2(i, carry):
                p = 2 * i
                idx_wait(pk_h, idxa, sema)
                idx_load(pk_h, dc, p + 1, idxb, semb)
                process_pair(xs, idxa)
                idx_wait(pk_h, idxb, semb)

                @pl.when(i < NPAIR // 2 - 1)
                def _():
                    idx_load(pk_h, dc, p + 2, idxa, sema)

                process_pair(xs, idxb)
                return carry

            lax.fori_loop(0, NPAIR // 2, pair2, 0)
            plsc.subcore_barrier()
            pltpu.sync_copy(acc.at[pl.ds(s * OUT_PT, OUT_PT)],
                            outs.at[pl.ds(s * OUT_PT, OUT_PT),
                                    pl.ds(dc * DC, DC)])
            plsc.subcore_barrier()

        if cout is not None:
            zero_acc()
            pltpu.sync_copy(ones_h, rows0)
            plsc.subcore_barrier()

            def cchunk(i, carry):
                idx_load(pk_h, 0, i, idxa, sema)
                idx_wait(pk_h, idxa, sema)
                scatter(idxa, 0, rows0, sems0)
                scatter(idxa, 1, rows0, sems1)
                scatter_wait(rows0, sems0)
                scatter_wait(rows0, sems1)
                return carry

            lax.fori_loop(0, NPAIR, cchunk, 0)
            plsc.subcore_barrier()
            pltpu.sync_copy(acc.at[pl.ds(s * OUT_PT, OUT_PT)],
                            cout.at[pl.ds(s * OUT_PT, OUT_PT), pl.ds(0, DC)])

    @pl.when(c == 0)
    def _():
        run_rel(xu, pk_ui, sui, cui)

    @pl.when(c == 1)
    def _():
        run_rel(xi, pk_iu, siu, ciu)


def _make_sc(with_cnt):
    mesh = plsc.VectorSubcoreMesh(core_axis_name="c", subcore_axis_name="s")
    n_out = 2 + (2 if with_cnt else 0)
    out_type = tuple(jax.ShapeDtypeStruct((ACC_ROWS, D), jnp.float32)
                     for _ in range(n_out))
    scratch = [
        pltpu.VMEM((4 * CHUNK,), jnp.int32),
        pltpu.VMEM((4 * CHUNK,), jnp.int32),
        pltpu.VMEM((CHUNK, DC), jnp.float32),
        pltpu.VMEM((CHUNK, DC), jnp.float32),
        pltpu.VMEM_SHARED((ACC_ROWS, DC), jnp.float32),
        pltpu.SemaphoreType.DMA,
        pltpu.SemaphoreType.DMA,
        pltpu.SemaphoreType.DMA,
        pltpu.SemaphoreType.DMA,
        pltpu.SemaphoreType.DMA,
        pltpu.SemaphoreType.DMA,
    ]
    return pl.kernel(functools.partial(_sc_body, with_cnt),
                     out_type=out_type, mesh=mesh, scratch_types=scratch,
                     compiler_params=pltpu.CompilerParams(
                         use_tc_tiling_on_sc=False,
                         internal_scratch_in_bytes=128 * 1024))


_sc_l0 = _make_sc(True)
_sc_l1 = _make_sc(False)


BLK = 512
GRID = (N + BLK - 1) // BLK


def _combine_body(s_ref, cnt_ref, x_ref, wl_ref, bl_ref, wr_ref, o_ref):
    cnt = jnp.maximum(cnt_ref[...][:, 0:1], 1.0)
    aggr = s_ref[...] / cnt
    out = (jnp.dot(aggr, wl_ref[...], preferred_element_type=jnp.float32)
           + bl_ref[...][None, :]
           + jnp.dot(x_ref[...], wr_ref[...], preferred_element_type=jnp.float32))
    o_ref[...] = jnp.maximum(out, 0.0)


def _combine(s, cnt32, x, wl, bl, wr):
    return pl.pallas_call(
        _combine_body,
        grid=(GRID,),
        in_specs=[pl.BlockSpec((BLK, D), lambda i: (i, 0)),
                  pl.BlockSpec((BLK, D), lambda i: (i, 0)),
                  pl.BlockSpec((BLK, D), lambda i: (i, 0)),
                  pl.BlockSpec((D, D), lambda i: (0, 0)),
                  pl.BlockSpec((D,), lambda i: (0,)),
                  pl.BlockSpec((D, D), lambda i: (0, 0))],
        out_specs=pl.BlockSpec((BLK, D), lambda i: (i, 0)),
        out_shape=jax.ShapeDtypeStruct((N, D), jnp.float32),
    )(s, cnt32, x, wl, bl, wr)


def _readout_body(xu_ref, xi_ref, wn_ref, bn_ref, su_ref, si_ref):
    i = pl.program_id(0)
    rows = i * BLK + lax.broadcasted_iota(jnp.int32, (BLK, 1), 0)
    mask = rows < N
    bn = bn_ref[...][None, :]
    hu = jnp.maximum(jnp.dot(xu_ref[...], wn_ref[...],
                             preferred_element_type=jnp.float32) + bn, 0.0)
    hi = jnp.maximum(jnp.dot(xi_ref[...], wn_ref[...],
                             preferred_element_type=jnp.float32) + bn, 0.0)
    su = jnp.sum(jnp.where(mask, hu, 0.0), axis=0, keepdims=True)
    si = jnp.sum(jnp.where(mask, hi, 0.0), axis=0, keepdims=True)

    @pl.when(i == 0)
    def _():
        su_ref[...] = jnp.zeros_like(su_ref)
        si_ref[...] = jnp.zeros_like(si_ref)

    su_ref[...] += su
    si_ref[...] += si


def _readout_sums(xu, xi, wn, bn):
    return pl.pallas_call(
        _readout_body,
        grid=(GRID,),
        in_specs=[pl.BlockSpec((BLK, D), lambda i: (i, 0)),
                  pl.BlockSpec((BLK, D), lambda i: (i, 0)),
                  pl.BlockSpec((D, D), lambda i: (0, 0)),
                  pl.BlockSpec((D,), lambda i: (0,))],
        out_specs=[pl.BlockSpec((1, D), lambda i: (0, 0)),
                   pl.BlockSpec((1, D), lambda i: (0, 0))],
        out_shape=[jax.ShapeDtypeStruct((1, D), jnp.float32),
                   jax.ShapeDtypeStruct((1, D), jnp.float32)],
    )(xu, xi, wn, bn)


def _final_body(su_ref, si_ref, wf_ref, bf_ref, o_ref):
    fu = su_ref[...] / float(N)
    fi = si_ref[...] / float(N)
    z = (jnp.dot(fu, wf_ref[0:D, :], preferred_element_type=jnp.float32)
         + jnp.dot(fi, wf_ref[D:2 * D, :], preferred_element_type=jnp.float32)
         + bf_ref[...][None, :])
    z = z - jnp.max(z, axis=1, keepdims=True)
    e = jnp.exp(z)
    o_ref[...] = e / jnp.sum(e, axis=1, keepdims=True)


def _final(su, si, wf, bf):
    return pl.pallas_call(
        _final_body,
        out_shape=jax.ShapeDtypeStruct((1, wf.shape[1]), jnp.float32),
    )(su, si, wf, bf)


def _pad_edges(ei):
    npair = NCHUNK // 2
    src = jnp.concatenate([ei[0], jnp.zeros((EPAD - E,), jnp.int32)])
    srcdc = (src[None, :] * NSLICE
             + jnp.arange(NSLICE, dtype=jnp.int32)[:, None])
    srcdc = srcdc.reshape(NSLICE, NSUB, npair, 2 * CHUNK)
    dst = jnp.concatenate([ei[1], jnp.full((EPAD - E,), N, jnp.int32)])
    dst = jnp.broadcast_to(dst.reshape(1, NSUB, npair, 2 * CHUNK),
                           srcdc.shape)
    return jnp.stack([srcdc, dst], axis=3).reshape(-1)


def _flat(x):
    return x.reshape(N * NSLICE, DC)


def kernel(x_user, x_item, edge_index_user_item, edge_index_item_user,
           l0_ui_Wl, l0_ui_bl, l0_ui_Wr, l0_iu_Wl, l0_iu_bl, l0_iu_Wr,
           l1_ui_Wl, l1_ui_bl, l1_ui_Wr, l1_iu_Wl, l1_iu_bl, l1_iu_Wr,
           Wn, bn, Wf, bf):
    pk_ui = _pad_edges(edge_index_user_item)
    pk_iu = _pad_edges(edge_index_item_user)
    zeros_h = jnp.zeros((ACC_ROWS, DC), jnp.float32)
    ones_h = jnp.ones((CHUNK, DC), jnp.float32)

    sui, siu, cnt_item32, cnt_user32 = _sc_l0(
        _flat(x_user), pk_ui, _flat(x_item), pk_iu, zeros_h, ones_h)

    xi1 = _combine(sui, cnt_item32, x_item, l0_ui_Wl, l0_ui_bl, l0_ui_Wr)
    xu1 = _combine(siu, cnt_user32, x_user, l0_iu_Wl, l0_iu_bl, l0_iu_Wr)

    sui1, siu1 = _sc_l1(
        _flat(xu1), pk_ui, _flat(xi1), pk_iu, zeros_h, ones_h)

    xi2 = _combine(sui1, cnt_item32, xi1, l1_ui_Wl, l1_ui_bl, l1_ui_Wr)
    xu2 = _combine(siu1, cnt_user32, xu1, l1_iu_Wl, l1_iu_bl, l1_iu_Wr)

    su, si = _readout_sums(xu2, xi2, Wn, bn)
    out = _final(su, si, Wf, bf)
    return jnp.reshape(out, (bf.shape[0],))
